# Initial kernel scaffold; baseline (speedup 1.0000x reference)
import jax
import jax.numpy as jnp
from jax.experimental import pallas as pl


def kernel(user_active_degree, is_live_streamer, is_video_author, video_type, tag, category_id, numeric_features, caption_embedding, emb_user_active_degree, emb_is_live_streamer, emb_is_video_author, emb_video_type, emb_tag, emb_category_id, W1, b1, W2, b2, ln_gamma, ln_beta, numeric_mean, numeric_std):
    raise NotImplementedError("write your pallas kernel here")


# speedup vs baseline: 2.9416x; 2.9416x over previous
import functools

import jax
import jax.numpy as jnp
from jax import lax
from jax.experimental import pallas as pl
from jax.experimental.pallas import tpu as pltpu
from jax.experimental.pallas import tpu_sc as plsc

B = 16384
EMB_DIM = 16
N_FEAT = 6
N_NUMERIC = 14
TEXT_DIM = 128
OUT_DIM = N_FEAT * EMB_DIM + N_NUMERIC + 64

NC = 2
NS = 16
NW = NC * NS
B_PER_W = B // NW
CHUNK = 128
N_CHUNKS = B_PER_W // CHUNK


def _sc_body(i0, i1, i2, i3, i4, i5, t0, t1, t2, t3, t4, t5,
             o0, o1, o2, o3, o4, o5,
             tab_v0, tab_v1, tab_v2, tab_v3, tab_v4, tab_v5,
             idx_v0, idx_v1, idx_v2, idx_v3, idx_v4, idx_v5,
             e_v0, e_v1, e_v2, e_v3, e_v4, e_v5):
    idx_h = (i0, i1, i2, i3, i4, i5)
    tab_h = (t0, t1, t2, t3, t4, t5)
    out_h = (o0, o1, o2, o3, o4, o5)
    tab_v = (tab_v0, tab_v1, tab_v2, tab_v3, tab_v4, tab_v5)
    idx_v = (idx_v0, idx_v1, idx_v2, idx_v3, idx_v4, idx_v5)
    e_v = (e_v0, e_v1, e_v2, e_v3, e_v4, e_v5)

    wid = lax.axis_index("s") * NC + lax.axis_index("c")
    base0 = wid * B_PER_W

    for k in range(N_FEAT):
        pltpu.sync_copy(tab_h[k], tab_v[k])

    lane = lax.iota(jnp.int32, EMB_DIM)
    lane16 = lane * EMB_DIM

    def chunk_body(c, _):
        base = base0 + c * CHUNK
        for k in range(N_FEAT):
            pltpu.sync_copy(idx_h[k].at[pl.ds(base, CHUNK)], idx_v[k])

        def group_body(g, _):
            dst_base = g * (16 * EMB_DIM) + lane16
            for k in range(N_FEAT):
                idx16 = idx_v[k][pl.ds(g * 16, 16)]
                src_base = idx16 * EMB_DIM
                for col in range(EMB_DIM):
                    vals = plsc.load_gather(tab_v[k], [src_base + col])
                    plsc.store_scatter(e_v[k], [dst_base + col], vals)
            return 0

        lax.fori_loop(0, CHUNK // 16, group_body, 0)
        for k in range(N_FEAT):
            pltpu.sync_copy(
                e_v[k], out_h[k].at[pl.ds(base * EMB_DIM, CHUNK * EMB_DIM)])
        return 0

    lax.fori_loop(0, N_CHUNKS, chunk_body, 0)


def _sc_gather(idxs, tables):
    mesh = plsc.VectorSubcoreMesh(core_axis_name="c", subcore_axis_name="s")
    scratch = ([pltpu.VMEM((int(t.size),), jnp.float32) for t in tables]
               + [pltpu.VMEM((CHUNK,), jnp.int32)] * N_FEAT
               + [pltpu.VMEM((CHUNK * EMB_DIM,), jnp.float32)] * N_FEAT)
    out_t = [jax.ShapeDtypeStruct((B * EMB_DIM,), jnp.float32)] * N_FEAT
    k = functools.partial(
        pl.kernel, mesh=mesh, out_type=out_t, scratch_types=scratch,
        compiler_params=pltpu.CompilerParams(use_tc_tiling_on_sc=False,
                                             needs_layout_passes=False),
    )(_sc_body)
    flat_tables = [t.reshape(-1) for t in tables]
    outs = k(*idxs, *flat_tables)
    return [o.reshape(B, EMB_DIM) for o in outs]


def _tc_body(e0, e1, e2, e3, e4, e5, num_ref, cap_ref, w1_ref, b1_ref,
             w2_ref, b2_ref, g_ref, beta_ref, mean_ref, std_ref, out_ref):
    num = (num_ref[...] - mean_ref[...]) / (std_ref[...] + 1e-8)
    h = jnp.dot(cap_ref[...], w1_ref[...],
                preferred_element_type=jnp.float32) + b1_ref[...]
    h = jnp.maximum(h, 0.0)
    h = jnp.dot(h, w2_ref[...], preferred_element_type=jnp.float32) + b2_ref[...]
    mu = jnp.mean(h, axis=-1, keepdims=True)
    var = jnp.mean((h - mu) * (h - mu), axis=-1, keepdims=True)
    t = (h - mu) * lax.rsqrt(var + 1e-5) * g_ref[...] + beta_ref[...]
    out_ref[...] = jnp.concatenate(
        [e0[...], e1[...], e2[...], e3[...], e4[...], e5[...], num, t],
        axis=-1)


def _tc_encode(embs, numeric, caption, w1, b1, w2, b2, g, beta, mean, std):
    bb = 2048
    grid = (B // bb,)
    full = lambda i: (0, 0)
    row = lambda i: (i, 0)
    return pl.pallas_call(
        _tc_body,
        grid=grid,
        in_specs=[pl.BlockSpec((bb, EMB_DIM), row)] * N_FEAT + [
            pl.BlockSpec((bb, N_NUMERIC), row),
            pl.BlockSpec((bb, TEXT_DIM), row),
            pl.BlockSpec((TEXT_DIM, 128), full),
            pl.BlockSpec((1, 128), full),
            pl.BlockSpec((128, 64), full),
            pl.BlockSpec((1, 64), full),
            pl.BlockSpec((1, 64), full),
            pl.BlockSpec((1, 64), full),
            pl.BlockSpec((1, N_NUMERIC), full),
            pl.BlockSpec((1, N_NUMERIC), full),
        ],
        out_specs=pl.BlockSpec((bb, OUT_DIM), row),
        out_shape=jax.ShapeDtypeStruct((B, OUT_DIM), jnp.float32),
    )(*embs, numeric, caption, w1, b1.reshape(1, -1), w2, b2.reshape(1, -1),
      g.reshape(1, -1), beta.reshape(1, -1), mean.reshape(1, -1),
      std.reshape(1, -1))


def kernel(user_active_degree, is_live_streamer, is_video_author, video_type,
           tag, category_id, numeric_features, caption_embedding,
           emb_user_active_degree, emb_is_live_streamer, emb_is_video_author,
           emb_video_type, emb_tag, emb_category_id,
           W1, b1, W2, b2, ln_gamma, ln_beta, numeric_mean, numeric_std):
    idxs = [x.astype(jnp.int32) for x in
            (user_active_degree, is_live_streamer, is_video_author,
             video_type, tag, category_id)]
    tables = (emb_user_active_degree, emb_is_live_streamer,
              emb_is_video_author, emb_video_type, emb_tag, emb_category_id)
    embs = _sc_gather(idxs, tables)
    return _tc_encode(embs, numeric_features, caption_embedding, W1, b1, W2,
                      b2, ln_gamma, ln_beta, numeric_mean, numeric_std)


# speedup vs baseline: 3.9114x; 1.3297x over previous
import functools

import jax
import jax.numpy as jnp
from jax import lax
from jax.experimental import pallas as pl
from jax.experimental.pallas import tpu as pltpu
from jax.experimental.pallas import tpu_sc as plsc

B = 16384
EMB_DIM = 16
N_FEAT = 6
N_NUMERIC = 14
TEXT_DIM = 128
PACK = 128
OUT_DIM = N_FEAT * EMB_DIM + N_NUMERIC + 64

NC = 2
NS = 16
NW = NC * NS
B_PER_W = B // NW
CHUNK = 128
N_CHUNKS = B_PER_W // CHUNK


def _sc_body(i0, i1, i2, i3, i4, i5, t0, t1, t2, t3, t4, t5, out_h,
             tab_v0, tab_v1, tab_v2, tab_v3, tab_v4, tab_v5,
             idx_v0, idx_v1, idx_v2, idx_v3, idx_v4, idx_v5,
             pack_a, pack_b, sem):
    idx_h = (i0, i1, i2, i3, i4, i5)
    tab_h = (t0, t1, t2, t3, t4, t5)
    tab_v = (tab_v0, tab_v1, tab_v2, tab_v3, tab_v4, tab_v5)
    idx_v = (idx_v0, idx_v1, idx_v2, idx_v3, idx_v4, idx_v5)
    packs = (pack_a, pack_b)

    wid = lax.axis_index("s") * NC + lax.axis_index("c")
    base0 = wid * B_PER_W

    loads = [pltpu.async_copy(tab_h[k], tab_v[k], sem) for k in range(N_FEAT)]
    loads += [pltpu.async_copy(idx_h[k].at[pl.ds(base0, B_PER_W)], idx_v[k],
                               sem) for k in range(N_FEAT)]
    for cp in loads:
        cp.wait()

    lane = lax.iota(jnp.int32, 16)
    dst_lane = lane * PACK

    def do_chunk(c, buf):
        for g in range(CHUNK // 16):
            dst_g = dst_lane + g * (16 * PACK)
            for k in range(N_FEAT):
                idx16 = idx_v[k][pl.ds(c * CHUNK + g * 16, 16)]
                src = idx16 * EMB_DIM
                dst = dst_g + k * EMB_DIM
                for col in range(EMB_DIM):
                    vals = plsc.load_gather(tab_v[k], [src + col])
                    plsc.store_scatter(buf, [dst + col], vals)

    def pair_body(p, _):
        c0 = p * 2
        do_chunk(c0, pack_a)
        cp_a = pltpu.async_copy(
            pack_a,
            out_h.at[pl.ds((base0 + c0 * CHUNK) * PACK, CHUNK * PACK)], sem)
        do_chunk(c0 + 1, pack_b)
        cp_b = pltpu.async_copy(
            pack_b,
            out_h.at[pl.ds((base0 + (c0 + 1) * CHUNK) * PACK, CHUNK * PACK)],
            sem)
        cp_a.wait()
        cp_b.wait()
        return 0

    lax.fori_loop(0, N_CHUNKS // 2, pair_body, 0)


def _sc_gather(idxs, tables):
    mesh = plsc.VectorSubcoreMesh(core_axis_name="c", subcore_axis_name="s")
    scratch = ([pltpu.VMEM((int(t.size),), jnp.float32) for t in tables]
               + [pltpu.VMEM((B_PER_W,), jnp.int32)] * N_FEAT
               + [pltpu.VMEM((CHUNK * PACK,), jnp.float32)] * 2
               + [pltpu.SemaphoreType.DMA])
    k = functools.partial(
        pl.kernel, mesh=mesh,
        out_type=jax.ShapeDtypeStruct((B * PACK,), jnp.float32),
        scratch_types=scratch,
        compiler_params=pltpu.CompilerParams(use_tc_tiling_on_sc=False,
                                             needs_layout_passes=False),
    )(_sc_body)
    flat_tables = [t.reshape(-1) for t in tables]
    return k(*idxs, *flat_tables).reshape(B, PACK)


def _tc_body(e_ref, num_ref, cap_ref, w1_ref, b1_ref,
             w2_ref, b2_ref, g_ref, beta_ref, mean_ref, std_ref, out_ref):
    num = (num_ref[...] - mean_ref[...]) / (std_ref[...] + 1e-8)
    h = jnp.dot(cap_ref[...], w1_ref[...],
                preferred_element_type=jnp.float32) + b1_ref[...]
    h = jnp.maximum(h, 0.0)
    h = jnp.dot(h, w2_ref[...], preferred_element_type=jnp.float32) + b2_ref[...]
    mu = jnp.mean(h, axis=-1, keepdims=True)
    var = jnp.mean((h - mu) * (h - mu), axis=-1, keepdims=True)
    t = (h - mu) * lax.rsqrt(var + 1e-5) * g_ref[...] + beta_ref[...]
    out_ref[...] = jnp.concatenate(
        [e_ref[:, :N_FEAT * EMB_DIM], num, t], axis=-1)


def _tc_encode(packed, numeric, caption, w1, b1, w2, b2, g, beta, mean, std):
    bb = 2048
    grid = (B // bb,)
    full = lambda i: (0, 0)
    row = lambda i: (i, 0)
    return pl.pallas_call(
        _tc_body,
        grid=grid,
        in_specs=[
            pl.BlockSpec((bb, PACK), row),
            pl.BlockSpec((bb, N_NUMERIC), row),
            pl.BlockSpec((bb, TEXT_DIM), row),
            pl.BlockSpec((TEXT_DIM, 128), full),
            pl.BlockSpec((1, 128), full),
            pl.BlockSpec((128, 64), full),
            pl.BlockSpec((1, 64), full),
            pl.BlockSpec((1, 64), full),
            pl.BlockSpec((1, 64), full),
            pl.BlockSpec((1, N_NUMERIC), full),
            pl.BlockSpec((1, N_NUMERIC), full),
        ],
        out_specs=pl.BlockSpec((bb, OUT_DIM), row),
        out_shape=jax.ShapeDtypeStruct((B, OUT_DIM), jnp.float32),
    )(packed, numeric, caption, w1, b1.reshape(1, -1), w2, b2.reshape(1, -1),
      g.reshape(1, -1), beta.reshape(1, -1), mean.reshape(1, -1),
      std.reshape(1, -1))


def kernel(user_active_degree, is_live_streamer, is_video_author, video_type,
           tag, category_id, numeric_features, caption_embedding,
           emb_user_active_degree, emb_is_live_streamer, emb_is_video_author,
           emb_video_type, emb_tag, emb_category_id,
           W1, b1, W2, b2, ln_gamma, ln_beta, numeric_mean, numeric_std):
    idxs = [x.astype(jnp.int32) for x in
            (user_active_degree, is_live_streamer, is_video_author,
             video_type, tag, category_id)]
    tables = (emb_user_active_degree, emb_is_live_streamer,
              emb_is_video_author, emb_video_type, emb_tag, emb_category_id)
    packed = _sc_gather(idxs, tables)
    return _tc_encode(packed, numeric_features, caption_embedding, W1, b1, W2,
                      b2, ln_gamma, ln_beta, numeric_mean, numeric_std)


# speedup vs baseline: 4.4652x; 1.1416x over previous
import functools

import jax
import jax.numpy as jnp
from jax import lax
from jax.experimental import pallas as pl
from jax.experimental.pallas import tpu as pltpu
from jax.experimental.pallas import tpu_sc as plsc

B = 16384
EMB_DIM = 16
N_NUMERIC = 14
TEXT_DIM = 128
PACK = 128
TAG_COL = 64
CAT_COL = 80
SIDE_COL = 96
OUT_DIM = 6 * EMB_DIM + N_NUMERIC + 64

NC = 2
NS = 16
NW = NC * NS
B_PER_W = B // NW
CHUNK = 128
N_CHUNKS = B_PER_W // CHUNK


def _sc_body(i0, i1, i2, i3, i4, i5, tab_tag_h, tab_cat_h, out_h,
             tag_v, cat_v,
             idx_v0, idx_v1, idx_v2, idx_v3, idx_v4, idx_v5,
             pack_a, pack_b, sem):
    idx_h = (i0, i1, i2, i3, i4, i5)
    idx_v = (idx_v0, idx_v1, idx_v2, idx_v3, idx_v4, idx_v5)

    wid = lax.axis_index("s") * NC + lax.axis_index("c")
    base0 = wid * B_PER_W

    loads = [pltpu.async_copy(tab_tag_h, tag_v, sem),
             pltpu.async_copy(tab_cat_h, cat_v, sem)]
    loads += [pltpu.async_copy(idx_h[k].at[pl.ds(base0, B_PER_W)], idx_v[k],
                               sem) for k in range(6)]
    for cp in loads:
        cp.wait()

    lane = lax.iota(jnp.int32, 16)
    dst_lane = lane * PACK

    def do_chunk(c, buf):
        for g in range(CHUNK // 16):
            dst_g = dst_lane + g * (16 * PACK)
            for tab, k, col0 in ((tag_v, 4, TAG_COL), (cat_v, 5, CAT_COL)):
                idx16 = idx_v[k][pl.ds(c * CHUNK + g * 16, 16)]
                src = idx16 * EMB_DIM
                dst = dst_g + col0
                for col in range(EMB_DIM):
                    vals = plsc.load_gather(tab, [src + col])
                    plsc.store_scatter(buf, [dst + col], vals)
            for k in range(4):
                idx16 = idx_v[k][pl.ds(c * CHUNK + g * 16, 16)]
                plsc.store_scatter(buf, [dst_g + (SIDE_COL + k)],
                                   idx16.astype(jnp.float32))

    def pair_body(p, _):
        c0 = p * 2
        do_chunk(c0, pack_a)
        cp_a = pltpu.async_copy(
            pack_a,
            out_h.at[pl.ds((base0 + c0 * CHUNK) * PACK, CHUNK * PACK)], sem)
        do_chunk(c0 + 1, pack_b)
        cp_b = pltpu.async_copy(
            pack_b,
            out_h.at[pl.ds((base0 + (c0 + 1) * CHUNK) * PACK, CHUNK * PACK)],
            sem)
        cp_a.wait()
        cp_b.wait()
        return 0

    lax.fori_loop(0, N_CHUNKS // 2, pair_body, 0)


def _sc_gather(idxs, tab_tag, tab_cat):
    mesh = plsc.VectorSubcoreMesh(core_axis_name="c", subcore_axis_name="s")
    scratch = ([pltpu.VMEM((int(tab_tag.size),), jnp.float32),
                pltpu.VMEM((int(tab_cat.size),), jnp.float32)]
               + [pltpu.VMEM((B_PER_W,), jnp.int32)] * 6
               + [pltpu.VMEM((CHUNK * PACK,), jnp.float32)] * 2
               + [pltpu.SemaphoreType.DMA])
    k = functools.partial(
        pl.kernel, mesh=mesh,
        out_type=jax.ShapeDtypeStruct((B * PACK,), jnp.float32),
        scratch_types=scratch,
        compiler_params=pltpu.CompilerParams(use_tc_tiling_on_sc=False,
                                             needs_layout_passes=False),
    )(_sc_body)
    return k(*idxs, tab_tag.reshape(-1), tab_cat.reshape(-1)).reshape(B, PACK)


def _tiny_lookup(side, tiny_ref, row0, vocab):
    e = tiny_ref[row0:row0 + 1, :]
    for v in range(1, vocab):
        e = jnp.where(side == float(v), tiny_ref[row0 + v:row0 + v + 1, :], e)
    return e


def _tc_body(packed_ref, tiny_ref, num_ref, cap_ref, w1_ref, b1_ref,
             w2_ref, b2_ref, g_ref, beta_ref, mean_ref, std_ref, out_ref):
    packed = packed_ref[...]
    tiny = [_tiny_lookup(packed[:, SIDE_COL + k:SIDE_COL + k + 1], tiny_ref,
                         r0, v)
            for k, (r0, v) in enumerate(((0, 4), (4, 2), (6, 2), (8, 2)))]
    num = (num_ref[...] - mean_ref[...]) / (std_ref[...] + 1e-8)
    h = jnp.dot(cap_ref[...], w1_ref[...],
                preferred_element_type=jnp.float32) + b1_ref[...]
    h = jnp.maximum(h, 0.0)
    h = jnp.dot(h, w2_ref[...], preferred_element_type=jnp.float32) + b2_ref[...]
    mu = jnp.mean(h, axis=-1, keepdims=True)
    var = jnp.mean((h - mu) * (h - mu), axis=-1, keepdims=True)
    t = (h - mu) * lax.rsqrt(var + 1e-5) * g_ref[...] + beta_ref[...]
    out_ref[...] = jnp.concatenate(
        tiny + [packed[:, TAG_COL:SIDE_COL], num, t], axis=-1)


def _tc_encode(packed, tiny_tabs, numeric, caption, w1, b1, w2, b2, g, beta,
               mean, std):
    bb = 2048
    grid = (B // bb,)
    full = lambda i: (0, 0)
    row = lambda i: (i, 0)
    return pl.pallas_call(
        _tc_body,
        grid=grid,
        in_specs=[
            pl.BlockSpec((bb, PACK), row),
            pl.BlockSpec((16, EMB_DIM), full),
            pl.BlockSpec((bb, N_NUMERIC), row),
            pl.BlockSpec((bb, TEXT_DIM), row),
            pl.BlockSpec((TEXT_DIM, 128), full),
            pl.BlockSpec((1, 128), full),
            pl.BlockSpec((128, 64), full),
            pl.BlockSpec((1, 64), full),
            pl.BlockSpec((1, 64), full),
            pl.BlockSpec((1, 64), full),
            pl.BlockSpec((1, N_NUMERIC), full),
            pl.BlockSpec((1, N_NUMERIC), full),
        ],
        out_specs=pl.BlockSpec((bb, OUT_DIM), row),
        out_shape=jax.ShapeDtypeStruct((B, OUT_DIM), jnp.float32),
    )(packed, tiny_tabs, numeric, caption, w1, b1.reshape(1, -1), w2,
      b2.reshape(1, -1), g.reshape(1, -1), beta.reshape(1, -1),
      mean.reshape(1, -1), std.reshape(1, -1))


def kernel(user_active_degree, is_live_streamer, is_video_author, video_type,
           tag, category_id, numeric_features, caption_embedding,
           emb_user_active_degree, emb_is_live_streamer, emb_is_video_author,
           emb_video_type, emb_tag, emb_category_id,
           W1, b1, W2, b2, ln_gamma, ln_beta, numeric_mean, numeric_std):
    idxs = [x.astype(jnp.int32) for x in
            (user_active_degree, is_live_streamer, is_video_author,
             video_type, tag, category_id)]
    packed = _sc_gather(idxs, emb_tag, emb_category_id)
    tiny_tabs = jnp.concatenate(
        [emb_user_active_degree, emb_is_live_streamer, emb_is_video_author,
         emb_video_type, jnp.zeros((6, EMB_DIM), jnp.float32)], axis=0)
    return _tc_encode(packed, tiny_tabs, numeric_features, caption_embedding,
                      W1, b1, W2, b2, ln_gamma, ln_beta, numeric_mean,
                      numeric_std)
